# Initial kernel scaffold; baseline (speedup 1.0000x reference)
import jax
import jax.numpy as jnp
from jax.experimental import pallas as pl


def kernel(rpn_cls_score, rpn_bbox_pred, base_feat, im_info, gt_boxes, num_boxes):
    raise NotImplementedError("write your pallas kernel here")


# speedup vs baseline: 16.9437x; 16.9437x over previous
import numpy as np
import jax
import jax.numpy as jnp
from jax import lax
from jax.experimental import pallas as pl
from jax.experimental.pallas import tpu as pltpu
from jax.experimental.pallas import tpu_sc as plsc

_FEAT_STRIDE = 16.0
_SCALES = np.array([8.0, 16.0, 32.0])
_RATIOS = np.array([0.5, 1.0, 2.0])

_NC, _NS, _L = 2, 16, 16


def _base_anchors(base_size=16):
    base = np.array([0.0, 0.0, base_size - 1.0, base_size - 1.0])
    w = base[2] - base[0] + 1.0
    h = base[3] - base[1] + 1.0
    xc = base[0] + 0.5 * (w - 1.0)
    yc = base[1] + 0.5 * (h - 1.0)
    size = w * h
    ws = np.round(np.sqrt(size / _RATIOS))
    hs = np.round(ws * _RATIOS)
    ra = np.stack([xc - 0.5 * (ws - 1), yc - 0.5 * (hs - 1),
                   xc + 0.5 * (ws - 1), yc + 0.5 * (hs - 1)], 1)
    out = []
    for a in ra:
        aw = a[2] - a[0] + 1.0
        ah = a[3] - a[1] + 1.0
        axc = a[0] + 0.5 * (aw - 1.0)
        ayc = a[1] + 0.5 * (ah - 1.0)
        ws2 = aw * _SCALES
        hs2 = ah * _SCALES
        out.append(np.stack([axc - 0.5 * (ws2 - 1), ayc - 0.5 * (hs2 - 1),
                             axc + 0.5 * (ws2 - 1), ayc + 0.5 * (hs2 - 1)], 1))
    return np.concatenate(out, 0).astype(np.float32)


_BASE = _base_anchors()
_EW = _BASE[:, 2] - _BASE[:, 0] + np.float32(1.0)
_EH = _BASE[:, 3] - _BASE[:, 1] + np.float32(1.0)
_AA = _EW * _EH
_ECX0 = _BASE[:, 0] + np.float32(0.5) * _EW
_ECY0 = _BASE[:, 1] + np.float32(0.5) * _EH

_ACON = np.zeros((16, 16), np.float32)
_ACON[0, :9] = _BASE[:, 0]
_ACON[1, :9] = _BASE[:, 1]
_ACON[2, :9] = _BASE[:, 2]
_ACON[3, :9] = _BASE[:, 3]
_ACON[4, :9] = _AA
_ACON[5, :9] = _ECX0
_ACON[6, :9] = _ECY0
_ACON[7, :9] = 1.0 / _EW.astype(np.float64)
_ACON[8, :9] = 1.0 / _EH.astype(np.float64)

_K_SM = np.float32(1.0 / 9.0)
_K_SM2 = np.float32(0.5 / 9.0)
_SIGHALF = np.float32(4.5)


def _sc_body(cls_hbm, prd_hbm, gtp_hbm, ac_hbm, out_hbm,
             gtp_v, ac_v, clsb, predb, mxv, agv, gacc, gmaxv, tmpv, resv,
             ihc_v, sum_v, shared, *sems):
    c = lax.axis_index("c")
    s = lax.axis_index("s")
    A = 9

    def bc(ref, idxs):
        return plsc.load_gather(
            ref, [jnp.full((16,), i, jnp.int32) for i in idxs])

    pltpu.sync_copy(ac_hbm, ac_v)
    pltpu.sync_copy(gtp_hbm.at[pl.ds(c * 4, 4)], gtp_v)

    descs = []
    for bl in range(4):
        bi = c * 4 + bl
        d1 = pltpu.async_copy(cls_hbm.at[bi, :, pl.ds(s * 4, 4), :],
                              clsb.at[bl], sems[2 * bl])
        d2 = pltpu.async_copy(prd_hbm.at[bi, :, pl.ds(s * 4, 4), :],
                              predb.at[bl], sems[2 * bl + 1])
        descs.append((d1, d2))

    iotf = lax.iota(jnp.int32, 16).astype(jnp.float32)
    shx = iotf * 16.0

    for bl in range(4):
        def init_body(gi, _):
            gacc[bl, gi] = jnp.full((16,), -1.0, jnp.float32)
            return 0
        lax.fori_loop(0, 32, init_body, 0)

    for bl in range(4):
        mrow = gtp_v[bl, 25, pl.ds(0, 16)]
        imh = mrow[0]
        imw = mrow[1]
        nb = mrow[2].astype(jnp.int32)

        def a_body(ai, _, bl=bl, imh=imh, imw=imw, nb=nb):
            bx1 = bc(ac_v, (0, ai))
            by1 = bc(ac_v, (1, ai))
            bx2 = bc(ac_v, (2, ai))
            by2 = bc(ac_v, (3, ai))
            aa = bc(ac_v, (4, ai))
            pxlo = bc(gtp_v, (bl, 26, ai))[0].astype(jnp.int32)
            pxhi = bc(gtp_v, (bl, 27, ai))[0].astype(jnp.int32)
            pylo = bc(gtp_v, (bl, 28, ai))[0].astype(jnp.int32)
            pyhi = bc(gtp_v, (bl, 29, ai))[0].astype(jnp.int32)
            rlo = jnp.clip(pylo - s * 4, 0, 4)
            rhi = jnp.clip(pyhi + 1 - s * 4, 0, 4)
            kxlo = pxlo // 16
            kxhi = pxhi // 16 + 1

            def s_prep(gi, _):
                sum_v[gi] = aa + bc(gtp_v, (bl, 4, gi))
                return 0
            lax.fori_loop(0, nb, s_prep, 0)

            def r_body(ri, _):
                r = rlo + ri
                yi = (s * 4 + r).astype(jnp.float32) * 16.0
                ay1 = by1 + yi
                ay2 = by2 + yi

                def h_prep(gi, _):
                    gy1 = bc(gtp_v, (bl, 1, gi))
                    gy2 = bc(gtp_v, (bl, 3, gi))
                    ihs = jnp.minimum(ay2, gy2) - jnp.maximum(ay1, gy1) + 1.0
                    ihc_v[gi] = jnp.maximum(ihs, 0.0)
                    return 0
                lax.fori_loop(0, nb, h_prep, 0)

                def k_body(ki, _):
                    kx = kxlo + ki
                    xs = (kx * 256).astype(jnp.float32)
                    ax1 = (shx + xs) + bx1
                    ax2 = (shx + xs) + bx2
                    inside = ((ax1 >= 0.0) & (ax2 < imw)
                              & (ay1 >= 0.0) & (ay2 < imh))

                    def g_body(gi, carry):
                        mx, ag = carry
                        gx1 = bc(gtp_v, (bl, 0, gi))
                        gx2 = bc(gtp_v, (bl, 2, gi))
                        iw = jnp.minimum(ax2, gx2) - jnp.maximum(ax1, gx1) + 1.0
                        inter = jnp.maximum(iw, 0.0) * ihc_v[gi]
                        iou = inter / (sum_v[gi] - inter)
                        upd = iou > mx
                        ag = jnp.where(upd, gi, ag)
                        mx = jnp.where(upd, iou, mx)
                        gv = jnp.where(inside, iou, -1.0)
                        gacc[bl, gi] = jnp.maximum(gacc[bl, gi], gv)
                        return mx, ag

                    mx0 = jnp.full((16,), -1.0, jnp.float32)
                    ag0 = jnp.zeros((16,), jnp.int32)
                    mx, ag = lax.fori_loop(0, nb, g_body, (mx0, ag0))
                    mxv[bl, ai, r * 4 + kx] = mx
                    agv[bl, ai, r * 4 + kx] = ag
                    return 0

                lax.fori_loop(0, kxhi - kxlo, k_body, 0)
                return 0

            lax.fori_loop(0, rhi - rlo, r_body, 0)
            return 0

        lax.fori_loop(0, A, a_body, 0)

    pltpu.sync_copy(gacc, shared.at[s])
    plsc.subcore_barrier()

    def t_body(t, _):
        pltpu.sync_copy(shared.at[t], tmpv)
        for bl in range(4):
            def r_body(gi, _, bl=bl):
                gacc[bl, gi] = jnp.maximum(gacc[bl, gi], tmpv[bl, gi])
                return 0
            lax.fori_loop(0, 20, r_body, 0)
        return 0

    lax.fori_loop(0, 16, t_body, 0)

    for bl in range(4):
        def m_body(gi, _, bl=bl):
            m = jnp.max(gacc[bl, gi])
            m = jnp.where(m <= 0.0, jnp.float32(-2.0), m)
            gmaxv[bl, gi] = jnp.full((16,), m)
            return 0
        lax.fori_loop(0, 20, m_body, 0)

    nll_t = jnp.zeros((16,), jnp.float32)
    for bl in range(4):
        descs[bl][0].wait()
        descs[bl][1].wait()
        mrow = gtp_v[bl, 25, pl.ds(0, 16)]
        imh = mrow[0]
        imw = mrow[1]
        nb = mrow[2].astype(jnp.int32)
        blv = jnp.full((16,), bl, jnp.int32)

        def a_body(ai, carry, bl=bl, imh=imh, imw=imw, nb=nb, blv=blv):
            bx1 = bc(ac_v, (0, ai))
            by1 = bc(ac_v, (1, ai))
            bx2 = bc(ac_v, (2, ai))
            by2 = bc(ac_v, (3, ai))
            aa = bc(ac_v, (4, ai))
            ecx0 = bc(ac_v, (5, ai))
            ecy0 = bc(ac_v, (6, ai))
            iew = bc(ac_v, (7, ai))
            ieh = bc(ac_v, (8, ai))
            pxlo = bc(gtp_v, (bl, 26, ai))[0].astype(jnp.int32)
            pxhi = bc(gtp_v, (bl, 27, ai))[0].astype(jnp.int32)
            pylo = bc(gtp_v, (bl, 28, ai))[0].astype(jnp.int32)
            pyhi = bc(gtp_v, (bl, 29, ai))[0].astype(jnp.int32)
            rlo = jnp.clip(pylo - s * 4, 0, 4)
            rhi = jnp.clip(pyhi + 1 - s * 4, 0, 4)
            kxlo = pxlo // 16
            kxhi = pxhi // 16 + 1

            def s_prep(gi, _):
                sum_v[gi] = aa + bc(gtp_v, (bl, 4, gi))
                return 0
            lax.fori_loop(0, nb, s_prep, 0)

            def r_body(ri, carry1):
                r = rlo + ri
                yi = (s * 4 + r).astype(jnp.float32) * 16.0
                ay1 = by1 + yi
                ay2 = by2 + yi
                ecy = ecy0 + yi

                def h_prep(gi, _):
                    gy1 = bc(gtp_v, (bl, 1, gi))
                    gy2 = bc(gtp_v, (bl, 3, gi))
                    ihs = jnp.minimum(ay2, gy2) - jnp.maximum(ay1, gy1) + 1.0
                    ihc_v[gi] = jnp.maximum(ihs, 0.0)
                    return 0
                lax.fori_loop(0, nb, h_prep, 0)

                def k_body(ki, carry2):
                    nll, cnt = carry2
                    kx = kxlo + ki
                    xs = (kx * 256).astype(jnp.float32)
                    ax1 = (shx + xs) + bx1
                    ax2 = (shx + xs) + bx2
                    inside = ((ax1 >= 0.0) & (ax2 < imw)
                              & (ay1 >= 0.0) & (ay2 < imh))

                    mx = mxv[bl, ai, r * 4 + kx]
                    ag = agv[bl, ai, r * 4 + kx]

                    def g_body(gi, kp):
                        gx1 = bc(gtp_v, (bl, 0, gi))
                        gx2 = bc(gtp_v, (bl, 2, gi))
                        iw = jnp.minimum(ax2, gx2) - jnp.maximum(ax1, gx1) + 1.0
                        inter = jnp.maximum(iw, 0.0) * ihc_v[gi]
                        iou = inter / (sum_v[gi] - inter)
                        return kp | (iou == gmaxv[bl, gi])

                    keep = lax.fori_loop(0, nb, g_body,
                                         jnp.zeros((16,), jnp.bool_))
                    pos = inside & (keep | (mx >= 0.7))
                    neg = inside & (mx < 0.3) & jnp.logical_not(pos)
                    val = pos | neg

                    xo = kx * 16
                    s0 = clsb[bl, ai, r, pl.ds(xo, 16)]
                    s1 = clsb[bl, A + ai, r, pl.ds(xo, 16)]
                    mxs = jnp.maximum(s0, s1)
                    ad = jnp.abs(s0 - s1)
                    t = jnp.exp(-ad)
                    z = t / (t + 2.0)
                    u = z * z
                    sp = z * (2.0 + u * (0.6666667 + u * (0.4 + u * (0.2857143 + u * 0.22222222))))
                    snll = mxs - jnp.where(pos, s1, s0) + sp
                    nll = nll + jnp.where(val, snll, 0.0)
                    cnt = cnt + jnp.where(val, 1.0, 0.0)

                    npos = plsc.all_reduce_population_count(pos)[0]

                    @pl.when(npos > 0)
                    def _():
                        gcx = plsc.load_gather(gtp_v, [blv, jnp.full((16,), 5, jnp.int32), ag])
                        gcy = plsc.load_gather(gtp_v, [blv, jnp.full((16,), 6, jnp.int32), ag])
                        lgw = plsc.load_gather(gtp_v, [blv, jnp.full((16,), 7, jnp.int32) + ai, ag])
                        lgh = plsc.load_gather(gtp_v, [blv, jnp.full((16,), 16, jnp.int32) + ai, ag])
                        ecx = (shx + xs) + ecx0
                        t0 = (gcx - ecx) * iew
                        t1 = (gcy - ecy) * ieh
                        lbs = jnp.zeros((16,), jnp.float32)
                        for cc, tt in ((0, t0), (1, t1), (2, lgw), (3, lgh)):
                            pcc = predb[bl, ai * 4 + cc, r, pl.ds(xo, 16)]
                            d = pcc - tt
                            ad2 = jnp.abs(d)
                            lbs = lbs + jnp.where(ad2 < _K_SM, _SIGHALF * d * d, ad2 - _K_SM2)
                        resv[5 + bl] = resv[5 + bl] + jnp.where(pos, lbs, 0.0)

                    return nll, cnt

                return lax.fori_loop(0, kxhi - kxlo, k_body, carry1)

            return lax.fori_loop(0, rhi - rlo, r_body, carry)

        z16 = jnp.zeros((16,), jnp.float32)
        resv[5 + bl] = z16
        nll_t, cnt_i = lax.fori_loop(0, A, a_body, (nll_t, z16))
        resv[1 + bl] = cnt_i

    resv[0] = nll_t
    pltpu.sync_copy(resv, out_hbm.at[c * 16 + s])


def kernel(rpn_cls_score, rpn_bbox_pred, base_feat, im_info, gt_boxes, num_boxes):
    del base_feat
    b = rpn_cls_score.shape[0]
    g = gt_boxes.shape[1]

    gx1 = gt_boxes[:, :, 0]
    gy1 = gt_boxes[:, :, 1]
    gx2 = gt_boxes[:, :, 2]
    gy2 = gt_boxes[:, :, 3]
    gw = gx2 - gx1 + 1.0
    gh = gy2 - gy1 + 1.0
    ga = gw * gh
    gcx = gx1 + 0.5 * gw
    gcy = gy1 + 0.5 * gh
    ew = jnp.asarray(_EW)
    eh = jnp.asarray(_EH)
    lgw = jnp.log(gw[:, None, :] / ew[None, :, None])
    lgh = jnp.log(gh[:, None, :] / eh[None, :, None])
    top = jnp.stack([gx1, gy1, gx2, gy2, ga, gcx, gcy], 1)
    misc = jnp.concatenate(
        [im_info[:, :2], num_boxes.astype(jnp.float32)[:, None],
         jnp.zeros((b, g - 3), jnp.float32)], axis=1)[:, None, :]
    bx1 = jnp.asarray(_BASE[:, 0])[None, :]
    by1 = jnp.asarray(_BASE[:, 1])[None, :]
    bx2 = jnp.asarray(_BASE[:, 2])[None, :]
    by2 = jnp.asarray(_BASE[:, 3])[None, :]
    imh = im_info[:, 0:1]
    imw = im_info[:, 1:2]
    pxlo = jnp.clip(jnp.ceil(-bx1 / 16.0) + jnp.zeros_like(imw), 0.0, 64.0)
    pxhi = jnp.clip(jnp.ceil((imw - bx2) / 16.0) - 1.0, -1.0, 63.0)
    pylo = jnp.clip(jnp.ceil(-by1 / 16.0) + jnp.zeros_like(imh), 0.0, 64.0)
    pyhi = jnp.clip(jnp.ceil((imh - by2) / 16.0) - 1.0, -1.0, 63.0)
    bnd = jnp.stack([pxlo, pxhi, pylo, pyhi], 1)
    bnd = jnp.pad(bnd, ((0, 0), (0, 0), (0, g - 9)))
    gtp = jnp.concatenate([top, lgw, lgh, misc, bnd], axis=1)
    gtp = jnp.pad(gtp, ((0, 0), (0, 0), (0, 32 - g)))
    acon = jnp.asarray(_ACON)

    mesh = plsc.VectorSubcoreMesh(core_axis_name="c", subcore_axis_name="s",
                                  num_cores=_NC, num_subcores=_NS)
    run = pl.kernel(
        _sc_body,
        out_type=jax.ShapeDtypeStruct((32, 9, 16), jnp.float32),
        mesh=mesh,
        compiler_params=pltpu.CompilerParams(needs_layout_passes=False,
                                             use_tc_tiling_on_sc=False),
        scratch_types=[
            pltpu.VMEM((4, 30, 32), jnp.float32),
            pltpu.VMEM((16, 16), jnp.float32),
            pltpu.VMEM((4, 18, 4, 64), jnp.float32),
            pltpu.VMEM((4, 36, 4, 64), jnp.float32),
            pltpu.VMEM((4, 9, 16, 16), jnp.float32),
            pltpu.VMEM((4, 9, 16, 16), jnp.int32),
            pltpu.VMEM((4, 32, 16), jnp.float32),
            pltpu.VMEM((4, 32, 16), jnp.float32),
            pltpu.VMEM((4, 32, 16), jnp.float32),
            pltpu.VMEM((9, 16), jnp.float32),
            pltpu.VMEM((32, 16), jnp.float32),
            pltpu.VMEM((32, 16), jnp.float32),
            pltpu.VMEM_SHARED((16, 4, 32, 16), jnp.float32),
        ] + [pltpu.SemaphoreType.DMA] * 8,
    )
    o = run(rpn_cls_score, rpn_bbox_pred, gtp, acon)

    o = o.reshape(2, 16, 9, 16)
    nll_total = jnp.sum(o[:, :, 0, :])
    cnt = jnp.sum(o[:, :, 1:5, :], axis=(1, 3)).reshape(-1)
    boxs = jnp.sum(o[:, :, 5:9, :], axis=(1, 3)).reshape(-1)
    loss_cls = nll_total / jnp.maximum(jnp.sum(cnt), 1.0)
    loss_box = jnp.mean(boxs / jnp.maximum(cnt, 1.0))
    return (loss_cls, loss_box)
